# baseline (speedup 1.0000x reference)
import functools

import jax
import jax.numpy as jnp
from jax import lax
from jax.experimental import pallas as pl
from jax.experimental.pallas import tpu as pltpu
from jax.experimental.pallas import tpu_sc as plsc

N = 10000
NPAD = 10240
K = 30
KPAD = 32
F = 64
C = 128
BIG = 1e30
BIGI = 2**30


def _knn_body(ublk_ref, ufull_ref, sqt_ref, idx_ref, d_ref, iacc_ref, *, rows):
    pid = pl.program_id(0)
    xb = ublk_ref[...]
    xf = ufull_ref[...]
    sqb = jnp.sum(xb * xb, axis=1, keepdims=True)
    sqf = sqt_ref[...]
    cross = lax.dot_general(xb, xf, (((1,), (1,)), ((), ())),
                            preferred_element_type=jnp.float32)
    d = sqb + sqf - 2.0 * cross
    rid = pid * rows + lax.broadcasted_iota(jnp.int32, (rows, NPAD), 0)
    cid = lax.broadcasted_iota(jnp.int32, (rows, NPAD), 1)
    d = jnp.where((cid == rid) | (cid >= N), BIG, d)
    d_ref[...] = d
    iacc_ref[...] = jnp.zeros((rows, KPAD), jnp.int32)

    lane = lax.broadcasted_iota(jnp.int32, (rows, KPAD), 1)

    def body(t, _):
        dcur = d_ref[...]
        m = jnp.min(dcur, axis=1, keepdims=True)
        cand = jnp.where(dcur == m, cid, BIGI)
        ai = jnp.min(cand, axis=1, keepdims=True)
        iacc_ref[...] = jnp.where(lane == t, ai, iacc_ref[...])
        d_ref[...] = jnp.where(cand == ai, BIG, dcur)
        return 0

    lax.fori_loop(0, K, body, 0)
    idx_ref[...] = iacc_ref[...]


def _sq_body(u_ref, o_ref):
    ub = u_ref[...]
    o_ref[...] = jnp.sum(ub * ub, axis=1, keepdims=True)


def _node_sq(u, rows=2048):
    return pl.pallas_call(
        _sq_body,
        grid=(NPAD // rows,),
        in_specs=[pl.BlockSpec((rows, C), lambda i: (i, 0))],
        out_specs=pl.BlockSpec((rows, 1), lambda i: (i, 0)),
        out_shape=jax.ShapeDtypeStruct((NPAD, 1), jnp.float32),
    )(u)


def _knn_topk(u, rows=256):
    sqt = _node_sq(u).T
    grid = NPAD // rows
    return pl.pallas_call(
        functools.partial(_knn_body, rows=rows),
        grid=(grid,),
        in_specs=[
            pl.BlockSpec((rows, C), lambda i: (i, 0)),
            pl.BlockSpec((NPAD, C), lambda i: (0, 0)),
            pl.BlockSpec((1, NPAD), lambda i: (0, 0)),
        ],
        out_specs=pl.BlockSpec((rows, KPAD), lambda i: (i, 0)),
        out_shape=jax.ShapeDtypeStruct((NPAD, KPAD), jnp.int32),
        scratch_shapes=[
            pltpu.VMEM((rows, NPAD), jnp.float32),
            pltpu.VMEM((rows, KPAD), jnp.int32),
        ],
    )(u, u, sqt)


_NW = 32
_NODES_PW = NPAD // _NW
_NODE_CHUNK = 4
_NODE_STEPS = _NODES_PW // _NODE_CHUNK
_GROWS = _NODE_CHUNK * KPAD


def _leaky16(v):
    return jnp.maximum(v, 0.2 * v)


def _diff_kernel(idx_hbm, u_hbm, diff_hbm, idx_v, uown_v,
                 browa_v, browb_v, dbuf_v, sema, semb):
    wid = lax.axis_index("s") * 2 + lax.axis_index("c")
    base = wid * _NODES_PW
    pltpu.sync_copy(idx_hbm.at[pl.ds(base * KPAD, _NODES_PW * KPAD)], idx_v)
    pltpu.sync_copy(u_hbm.at[pl.ds(base, _NODES_PW)], uown_v)

    def issue(sc, buf, sem):
        return pltpu.async_copy(
            u_hbm.at[idx_v.at[pl.ds(sc * _GROWS, _GROWS)]], buf, sem)

    def compute(sc, buf):
        def node(n, _):
            own = [uown_v[sc * _NODE_CHUNK + n, pl.ds(16 * j, 16)]
                   for j in range(C // 16)]
            for k in range(KPAD):
                row = n * KPAD + k
                for j in range(C // 16):
                    dbuf_v[row, pl.ds(16 * j, 16)] = (
                        buf[row, pl.ds(16 * j, 16)] - own[j])
            return 0

        lax.fori_loop(0, _NODE_CHUNK, node, 0)
        pltpu.sync_copy(
            dbuf_v, diff_hbm.at[pl.ds((base + sc * _NODE_CHUNK) * KPAD,
                                      _GROWS)])

    issue(0, browa_v, sema)

    def pair(s2, _):
        sc0 = 2 * s2
        pltpu.make_async_copy(
            u_hbm.at[idx_v.at[pl.ds(sc0 * _GROWS, _GROWS)]], browa_v, sema
        ).wait()
        issue(sc0 + 1, browb_v, semb)
        compute(sc0, browa_v)
        pltpu.make_async_copy(
            u_hbm.at[idx_v.at[pl.ds((sc0 + 1) * _GROWS, _GROWS)]], browb_v,
            semb).wait()

        @pl.when(sc0 + 2 < _NODE_STEPS)
        def _():
            issue(sc0 + 2, browa_v, sema)

        compute(sc0 + 1, browb_v)
        return 0

    lax.fori_loop(0, _NODE_STEPS // 2, pair, 0)


def _gather_diff(idx_flat, u):
    return pl.kernel(
        _diff_kernel,
        out_type=jax.ShapeDtypeStruct((NPAD * KPAD, C), jnp.float32),
        mesh=_sc_mesh(),
        scratch_types=[
            pltpu.VMEM((_NODES_PW * KPAD,), jnp.int32),
            pltpu.VMEM((_NODES_PW, C), jnp.float32),
            pltpu.VMEM((_GROWS, C), jnp.float32),
            pltpu.VMEM((_GROWS, C), jnp.float32),
            pltpu.VMEM((_GROWS, C), jnp.float32),
            pltpu.SemaphoreType.DMA,
            pltpu.SemaphoreType.DMA,
        ],
    )(idx_flat, u)


@functools.lru_cache(maxsize=1)
def _sc_mesh():
    return plsc.VectorSubcoreMesh(core_axis_name="c", subcore_axis_name="s")


_MB = 64


def _msg_body(u_ref, diff_ref, w_ref, b_ref, h_ref):
    xi = u_ref[...]
    xi_rep = jnp.broadcast_to(xi[:, None, :], (_MB, KPAD, C))
    xi_rep = xi_rep.reshape(_MB * KPAD, C)
    m = jnp.concatenate([xi_rep, diff_ref[...]], axis=1)
    z = lax.dot_general(m, w_ref[...], (((1,), (0,)), ((), ())),
                        preferred_element_type=jnp.float32) + b_ref[...]
    z = z.reshape(_MB, KPAD, F)
    kio = lax.broadcasted_iota(jnp.int32, (_MB, KPAD, F), 1)
    z = jnp.where(kio < K, z, -BIG)
    h = jnp.max(z, axis=1)
    h = jnp.where(h >= 0, h, 0.2 * h)
    h_ref[...] = jnp.concatenate(
        [h, jnp.zeros((_MB, C - F), jnp.float32)], axis=1)


def _msg_max(u, diff, wfull, bias):
    grid = NPAD // _MB
    return pl.pallas_call(
        _msg_body,
        grid=(grid,),
        in_specs=[
            pl.BlockSpec((_MB, C), lambda i: (i, 0)),
            pl.BlockSpec((_MB * KPAD, C), lambda i: (i, 0)),
            pl.BlockSpec((2 * C, F), lambda i: (0, 0)),
            pl.BlockSpec((1, F), lambda i: (0, 0)),
        ],
        out_specs=pl.BlockSpec((_MB, C), lambda i: (i, 0)),
        out_shape=jax.ShapeDtypeStruct((NPAD, C), jnp.float32),
    )(u, diff, wfull, bias)


def _ab_body(u_ref, w_ref, b_ref, ab_ref):
    ab_ref[...] = lax.dot_general(
        u_ref[...], w_ref[...], (((1,), (0,)), ((), ())),
        preferred_element_type=jnp.float32) + b_ref[...]


def _node_ab(u, wpack, bpack, rows=2048):
    grid = NPAD // rows
    return pl.pallas_call(
        _ab_body,
        grid=(grid,),
        in_specs=[
            pl.BlockSpec((rows, C), lambda i: (i, 0)),
            pl.BlockSpec((C, 2 * F), lambda i: (0, 0)),
            pl.BlockSpec((1, 2 * F), lambda i: (0, 0)),
        ],
        out_specs=pl.BlockSpec((rows, 2 * F), lambda i: (i, 0)),
        out_shape=jax.ShapeDtypeStruct((NPAD, 2 * F), jnp.float32),
    )(u, wpack, bpack)


EPAD = 163840
_EDGES_PW = EPAD // _NW
_EDGE_CHUNK = 128
_EDGE_STEPS = _EDGES_PW // _EDGE_CHUNK


def _edge_kernel(e0_hbm, e1_hbm, pq_hbm, out_hbm,
                 e0_v, e1_v, a_v, b_v, prow_v, qrow_v, z_v, semp, semq):
    wid = lax.axis_index("s") * 2 + lax.axis_index("c")
    base = wid * _EDGES_PW
    pltpu.sync_copy(e0_hbm.at[pl.ds(base, _EDGES_PW)], e0_v)
    pltpu.sync_copy(e1_hbm.at[pl.ds(base, _EDGES_PW)], e1_v)

    def sort_group(g, _):
        v0 = e0_v[pl.ds(g * 16, 16)]
        v1 = e1_v[pl.ds(g * 16, 16)]
        a_v[pl.ds(g * 16, 16)] = jnp.minimum(v0, v1)
        b_v[pl.ds(g * 16, 16)] = jnp.maximum(v0, v1)
        return 0

    lax.fori_loop(0, _EDGES_PW // 16, sort_group, 0)

    def sub(sc, _):
        cp = pltpu.async_copy(
            pq_hbm.at[a_v.at[pl.ds(sc * _EDGE_CHUNK, _EDGE_CHUNK)]],
            prow_v, semp)
        cq = pltpu.async_copy(
            pq_hbm.at[b_v.at[pl.ds(sc * _EDGE_CHUNK, _EDGE_CHUNK)]],
            qrow_v, semq)
        cp.wait()
        cq.wait()

        def edge(e, _):
            for j in range(F // 16):
                p = prow_v[e, pl.ds(16 * j, 16)]
                q = qrow_v[e, pl.ds(F + 16 * j, 16)]
                z_v[e, pl.ds(16 * j, 16)] = _leaky16(p + q)
            return 0

        lax.fori_loop(0, _EDGE_CHUNK, edge, 0)
        pltpu.sync_copy(z_v, out_hbm.at[pl.ds(base + sc * _EDGE_CHUNK,
                                              _EDGE_CHUNK)])
        return 0

    lax.fori_loop(0, _EDGE_STEPS, sub, 0)


def _edge_gather(e0, e1, pq_arr):
    return pl.kernel(
        _edge_kernel,
        out_type=jax.ShapeDtypeStruct((EPAD, F), jnp.float32),
        mesh=_sc_mesh(),
        scratch_types=[
            pltpu.VMEM((_EDGES_PW,), jnp.int32),
            pltpu.VMEM((_EDGES_PW,), jnp.int32),
            pltpu.VMEM((_EDGES_PW,), jnp.int32),
            pltpu.VMEM((_EDGES_PW,), jnp.int32),
            pltpu.VMEM((_EDGE_CHUNK, 2 * F), jnp.float32),
            pltpu.VMEM((_EDGE_CHUNK, 2 * F), jnp.float32),
            pltpu.VMEM((_EDGE_CHUNK, F), jnp.float32),
            pltpu.SemaphoreType.DMA,
            pltpu.SemaphoreType.DMA,
        ],
    )(e0, e1, pq_arr)


def _fin_body(z_ref, w_ref, o_ref):
    wbb = w_ref[...]
    s = jnp.sum(z_ref[...] * wbb[:, :F], axis=1, keepdims=True) + wbb[:, F:]
    o_ref[...] = 1.0 / (1.0 + jnp.exp(-s))


def _finalize(z_arr, wbb, rows=4096):
    grid = EPAD // rows
    return pl.pallas_call(
        _fin_body,
        grid=(grid,),
        in_specs=[
            pl.BlockSpec((rows, F), lambda i: (i, 0)),
            pl.BlockSpec((1, F + 1), lambda i: (0, 0)),
        ],
        out_specs=pl.BlockSpec((rows, 1), lambda i: (i, 0)),
        out_shape=jax.ShapeDtypeStruct((EPAD, 1), jnp.float32),
    )(z_arr, wbb)


def kernel(x, edge_index, W1, b1, W2, b2, W3, b3, Wa, ba, Wb, bb):
    f32 = jnp.float32

    def pad_w(w):
        c = w.shape[0] // 2
        zc = jnp.zeros((C - c, F), f32)
        return jnp.concatenate([w[:c].astype(f32), zc,
                                w[c:].astype(f32), zc], axis=0)

    u = jnp.pad(x.astype(f32), ((0, NPAD - N), (0, 0)))
    for w_l, b_l in ((W1, b1), (W2, b2), (W3, b3)):
        idx = _knn_topk(u)
        diff = _gather_diff(idx.reshape(-1), u)
        u = _msg_max(u, diff, pad_w(w_l), b_l.astype(f32)[None, :])

    zr = jnp.zeros((C - F, F), f32)
    wpa = jnp.concatenate(
        [jnp.concatenate([Wa[:F].astype(f32), zr], axis=0),
         jnp.concatenate([Wa[F:].astype(f32), zr], axis=0)], axis=1)
    bpa = jnp.concatenate([ba, jnp.zeros_like(ba)])[None, :].astype(f32)
    pq_arr = _node_ab(u, wpa, bpa)

    e = edge_index.shape[1]
    e0 = jnp.pad(edge_index[0].astype(jnp.int32), (0, EPAD - e))
    e1 = jnp.pad(edge_index[1].astype(jnp.int32), (0, EPAD - e))
    z_arr = _edge_gather(e0, e1, pq_arr)
    wbb = jnp.concatenate([Wb[:, 0], bb]).astype(f32)[None, :]
    out = _finalize(z_arr, wbb)
    return out.reshape(-1)[:e]


# speedup vs baseline: 1.0259x; 1.0259x over previous
import functools

import jax
import jax.numpy as jnp
from jax import lax
from jax.experimental import pallas as pl
from jax.experimental.pallas import tpu as pltpu
from jax.experimental.pallas import tpu_sc as plsc

N = 10000
NPAD = 10240
K = 30
KPAD = 32
F = 64
C = 128
BIG = 1e30
BIGI = 2**30


def _knn_body(ublk_ref, ufull_ref, sqt_ref, idx_ref, d_ref, iacc_ref, *, rows):
    pid = pl.program_id(0)
    xb = ublk_ref[...]
    xf = ufull_ref[...]
    sqb = jnp.sum(xb * xb, axis=1, keepdims=True)
    sqf = sqt_ref[...]
    cross = lax.dot_general(xb, xf, (((1,), (1,)), ((), ())),
                            preferred_element_type=jnp.float32)
    d = sqb + sqf - 2.0 * cross
    rid = pid * rows + lax.broadcasted_iota(jnp.int32, (rows, NPAD), 0)
    cid = lax.broadcasted_iota(jnp.int32, (rows, NPAD), 1)
    d = jnp.where((cid == rid) | (cid >= N), BIG, d)
    d_ref[...] = d
    iacc_ref[...] = jnp.zeros((rows, KPAD), jnp.int32)

    lane = lax.broadcasted_iota(jnp.int32, (rows, KPAD), 1)

    def body(t, _):
        dcur = d_ref[...]
        m = jnp.min(dcur, axis=1, keepdims=True)
        cand = jnp.where(dcur == m, cid, BIGI)
        ai = jnp.min(cand, axis=1, keepdims=True)
        iacc_ref[...] = jnp.where(lane == t, ai, iacc_ref[...])
        d_ref[...] = jnp.where(cand == ai, BIG, dcur)
        return 0

    lax.fori_loop(0, K, body, 0)
    idx_ref[...] = iacc_ref[...]


def _sq_body(u_ref, o_ref):
    ub = u_ref[...]
    o_ref[...] = jnp.sum(ub * ub, axis=1, keepdims=True)


def _node_sq(u, rows=2048):
    return pl.pallas_call(
        _sq_body,
        grid=(NPAD // rows,),
        in_specs=[pl.BlockSpec((rows, C), lambda i: (i, 0))],
        out_specs=pl.BlockSpec((rows, 1), lambda i: (i, 0)),
        out_shape=jax.ShapeDtypeStruct((NPAD, 1), jnp.float32),
    )(u)


def _knn_topk(u, rows=512):
    sqt = _node_sq(u).T
    grid = NPAD // rows
    return pl.pallas_call(
        functools.partial(_knn_body, rows=rows),
        grid=(grid,),
        in_specs=[
            pl.BlockSpec((rows, C), lambda i: (i, 0)),
            pl.BlockSpec((NPAD, C), lambda i: (0, 0)),
            pl.BlockSpec((1, NPAD), lambda i: (0, 0)),
        ],
        out_specs=pl.BlockSpec((rows, KPAD), lambda i: (i, 0)),
        out_shape=jax.ShapeDtypeStruct((NPAD, KPAD), jnp.int32),
        scratch_shapes=[
            pltpu.VMEM((rows, NPAD), jnp.float32),
            pltpu.VMEM((rows, KPAD), jnp.int32),
        ],
    )(u, u, sqt)


_NW = 32
_NODES_PW = NPAD // _NW
_NODE_CHUNK = 4
_NODE_STEPS = _NODES_PW // _NODE_CHUNK
_GROWS = _NODE_CHUNK * KPAD


def _leaky16(v):
    return jnp.maximum(v, 0.2 * v)


def _diff_kernel(idx_hbm, u_hbm, diff_hbm, idx_v, uown_v,
                 browa_v, browb_v, dbuf_v, sema, semb):
    wid = lax.axis_index("s") * 2 + lax.axis_index("c")
    base = wid * _NODES_PW
    pltpu.sync_copy(idx_hbm.at[pl.ds(base * KPAD, _NODES_PW * KPAD)], idx_v)
    pltpu.sync_copy(u_hbm.at[pl.ds(base, _NODES_PW)], uown_v)

    def issue(sc, buf, sem):
        return pltpu.async_copy(
            u_hbm.at[idx_v.at[pl.ds(sc * _GROWS, _GROWS)]], buf, sem)

    def compute(sc, buf):
        def node(n, _):
            own = [uown_v[sc * _NODE_CHUNK + n, pl.ds(16 * j, 16)]
                   for j in range(C // 16)]
            for k in range(KPAD):
                row = n * KPAD + k
                for j in range(C // 16):
                    dbuf_v[row, pl.ds(16 * j, 16)] = (
                        buf[row, pl.ds(16 * j, 16)] - own[j])
            return 0

        lax.fori_loop(0, _NODE_CHUNK, node, 0)
        pltpu.sync_copy(
            dbuf_v, diff_hbm.at[pl.ds((base + sc * _NODE_CHUNK) * KPAD,
                                      _GROWS)])

    issue(0, browa_v, sema)

    def pair(s2, _):
        sc0 = 2 * s2
        pltpu.make_async_copy(
            u_hbm.at[idx_v.at[pl.ds(sc0 * _GROWS, _GROWS)]], browa_v, sema
        ).wait()
        issue(sc0 + 1, browb_v, semb)
        compute(sc0, browa_v)
        pltpu.make_async_copy(
            u_hbm.at[idx_v.at[pl.ds((sc0 + 1) * _GROWS, _GROWS)]], browb_v,
            semb).wait()

        @pl.when(sc0 + 2 < _NODE_STEPS)
        def _():
            issue(sc0 + 2, browa_v, sema)

        compute(sc0 + 1, browb_v)
        return 0

    lax.fori_loop(0, _NODE_STEPS // 2, pair, 0)


def _gather_diff(idx_flat, u):
    return pl.kernel(
        _diff_kernel,
        out_type=jax.ShapeDtypeStruct((NPAD * KPAD, C), jnp.float32),
        mesh=_sc_mesh(),
        scratch_types=[
            pltpu.VMEM((_NODES_PW * KPAD,), jnp.int32),
            pltpu.VMEM((_NODES_PW, C), jnp.float32),
            pltpu.VMEM((_GROWS, C), jnp.float32),
            pltpu.VMEM((_GROWS, C), jnp.float32),
            pltpu.VMEM((_GROWS, C), jnp.float32),
            pltpu.SemaphoreType.DMA,
            pltpu.SemaphoreType.DMA,
        ],
    )(idx_flat, u)


@functools.lru_cache(maxsize=1)
def _sc_mesh():
    return plsc.VectorSubcoreMesh(core_axis_name="c", subcore_axis_name="s")


_MB = 64


def _msg_body(u_ref, diff_ref, w_ref, b_ref, h_ref):
    xi = u_ref[...]
    xi_rep = jnp.broadcast_to(xi[:, None, :], (_MB, KPAD, C))
    xi_rep = xi_rep.reshape(_MB * KPAD, C)
    m = jnp.concatenate([xi_rep, diff_ref[...]], axis=1)
    z = lax.dot_general(m, w_ref[...], (((1,), (0,)), ((), ())),
                        preferred_element_type=jnp.float32) + b_ref[...]
    z = z.reshape(_MB, KPAD, F)
    kio = lax.broadcasted_iota(jnp.int32, (_MB, KPAD, F), 1)
    z = jnp.where(kio < K, z, -BIG)
    h = jnp.max(z, axis=1)
    h = jnp.where(h >= 0, h, 0.2 * h)
    h_ref[...] = jnp.concatenate(
        [h, jnp.zeros((_MB, C - F), jnp.float32)], axis=1)


def _msg_max(u, diff, wfull, bias):
    grid = NPAD // _MB
    return pl.pallas_call(
        _msg_body,
        grid=(grid,),
        in_specs=[
            pl.BlockSpec((_MB, C), lambda i: (i, 0)),
            pl.BlockSpec((_MB * KPAD, C), lambda i: (i, 0)),
            pl.BlockSpec((2 * C, F), lambda i: (0, 0)),
            pl.BlockSpec((1, F), lambda i: (0, 0)),
        ],
        out_specs=pl.BlockSpec((_MB, C), lambda i: (i, 0)),
        out_shape=jax.ShapeDtypeStruct((NPAD, C), jnp.float32),
    )(u, diff, wfull, bias)


def _ab_body(u_ref, w_ref, b_ref, ab_ref):
    ab_ref[...] = lax.dot_general(
        u_ref[...], w_ref[...], (((1,), (0,)), ((), ())),
        preferred_element_type=jnp.float32) + b_ref[...]


def _node_ab(u, wpack, bpack, rows=2048):
    grid = NPAD // rows
    return pl.pallas_call(
        _ab_body,
        grid=(grid,),
        in_specs=[
            pl.BlockSpec((rows, C), lambda i: (i, 0)),
            pl.BlockSpec((C, 2 * F), lambda i: (0, 0)),
            pl.BlockSpec((1, 2 * F), lambda i: (0, 0)),
        ],
        out_specs=pl.BlockSpec((rows, 2 * F), lambda i: (i, 0)),
        out_shape=jax.ShapeDtypeStruct((NPAD, 2 * F), jnp.float32),
    )(u, wpack, bpack)


EPAD = 163840
_EDGES_PW = EPAD // _NW
_EDGE_CHUNK = 128
_EDGE_STEPS = _EDGES_PW // _EDGE_CHUNK


def _edge_kernel(e0_hbm, e1_hbm, pq_hbm, out_hbm,
                 e0_v, e1_v, a_v, b_v, prow_v, qrow_v, z_v, semp, semq):
    wid = lax.axis_index("s") * 2 + lax.axis_index("c")
    base = wid * _EDGES_PW
    pltpu.sync_copy(e0_hbm.at[pl.ds(base, _EDGES_PW)], e0_v)
    pltpu.sync_copy(e1_hbm.at[pl.ds(base, _EDGES_PW)], e1_v)

    def sort_group(g, _):
        v0 = e0_v[pl.ds(g * 16, 16)]
        v1 = e1_v[pl.ds(g * 16, 16)]
        a_v[pl.ds(g * 16, 16)] = jnp.minimum(v0, v1)
        b_v[pl.ds(g * 16, 16)] = jnp.maximum(v0, v1)
        return 0

    lax.fori_loop(0, _EDGES_PW // 16, sort_group, 0)

    def sub(sc, _):
        cp = pltpu.async_copy(
            pq_hbm.at[a_v.at[pl.ds(sc * _EDGE_CHUNK, _EDGE_CHUNK)]],
            prow_v, semp)
        cq = pltpu.async_copy(
            pq_hbm.at[b_v.at[pl.ds(sc * _EDGE_CHUNK, _EDGE_CHUNK)]],
            qrow_v, semq)
        cp.wait()
        cq.wait()

        def edge(e, _):
            for j in range(F // 16):
                p = prow_v[e, pl.ds(16 * j, 16)]
                q = qrow_v[e, pl.ds(F + 16 * j, 16)]
                z_v[e, pl.ds(16 * j, 16)] = _leaky16(p + q)
            return 0

        lax.fori_loop(0, _EDGE_CHUNK, edge, 0)
        pltpu.sync_copy(z_v, out_hbm.at[pl.ds(base + sc * _EDGE_CHUNK,
                                              _EDGE_CHUNK)])
        return 0

    lax.fori_loop(0, _EDGE_STEPS, sub, 0)


def _edge_gather(e0, e1, pq_arr):
    return pl.kernel(
        _edge_kernel,
        out_type=jax.ShapeDtypeStruct((EPAD, F), jnp.float32),
        mesh=_sc_mesh(),
        scratch_types=[
            pltpu.VMEM((_EDGES_PW,), jnp.int32),
            pltpu.VMEM((_EDGES_PW,), jnp.int32),
            pltpu.VMEM((_EDGES_PW,), jnp.int32),
            pltpu.VMEM((_EDGES_PW,), jnp.int32),
            pltpu.VMEM((_EDGE_CHUNK, 2 * F), jnp.float32),
            pltpu.VMEM((_EDGE_CHUNK, 2 * F), jnp.float32),
            pltpu.VMEM((_EDGE_CHUNK, F), jnp.float32),
            pltpu.SemaphoreType.DMA,
            pltpu.SemaphoreType.DMA,
        ],
    )(e0, e1, pq_arr)


def _fin_body(z_ref, w_ref, o_ref):
    wbb = w_ref[...]
    s = jnp.sum(z_ref[...] * wbb[:, :F], axis=1, keepdims=True) + wbb[:, F:]
    o_ref[...] = 1.0 / (1.0 + jnp.exp(-s))


def _finalize(z_arr, wbb, rows=4096):
    grid = EPAD // rows
    return pl.pallas_call(
        _fin_body,
        grid=(grid,),
        in_specs=[
            pl.BlockSpec((rows, F), lambda i: (i, 0)),
            pl.BlockSpec((1, F + 1), lambda i: (0, 0)),
        ],
        out_specs=pl.BlockSpec((rows, 1), lambda i: (i, 0)),
        out_shape=jax.ShapeDtypeStruct((EPAD, 1), jnp.float32),
    )(z_arr, wbb)


def kernel(x, edge_index, W1, b1, W2, b2, W3, b3, Wa, ba, Wb, bb):
    f32 = jnp.float32

    def pad_w(w):
        c = w.shape[0] // 2
        zc = jnp.zeros((C - c, F), f32)
        return jnp.concatenate([w[:c].astype(f32), zc,
                                w[c:].astype(f32), zc], axis=0)

    u = jnp.pad(x.astype(f32), ((0, NPAD - N), (0, 0)))
    for w_l, b_l in ((W1, b1), (W2, b2), (W3, b3)):
        idx = _knn_topk(u)
        diff = _gather_diff(idx.reshape(-1), u)
        u = _msg_max(u, diff, pad_w(w_l), b_l.astype(f32)[None, :])

    zr = jnp.zeros((C - F, F), f32)
    wpa = jnp.concatenate(
        [jnp.concatenate([Wa[:F].astype(f32), zr], axis=0),
         jnp.concatenate([Wa[F:].astype(f32), zr], axis=0)], axis=1)
    bpa = jnp.concatenate([ba, jnp.zeros_like(ba)])[None, :].astype(f32)
    pq_arr = _node_ab(u, wpa, bpa)

    e = edge_index.shape[1]
    e0 = jnp.pad(edge_index[0].astype(jnp.int32), (0, EPAD - e))
    e1 = jnp.pad(edge_index[1].astype(jnp.int32), (0, EPAD - e))
    z_arr = _edge_gather(e0, e1, pq_arr)
    wbb = jnp.concatenate([Wb[:, 0], bb]).astype(f32)[None, :]
    out = _finalize(z_arr, wbb)
    return out.reshape(-1)[:e]
